# baseline (speedup 1.0000x reference)
import functools

import jax
import jax.numpy as jnp
from jax import lax
from jax.experimental import pallas as pl
from jax.experimental.pallas import tpu as pltpu
from jax.experimental.pallas import tpu_sc as plsc

_NC = 2
_NS = 16
_NW = _NC * _NS
_L = 16
_S = 32


_DCH = 6
_SCH = 10


def _issue_pass(tab_hbm, idx_v, out_hbm, out_base, idx_base, buf_v, sem,
                dsem):
    lanes = lax.iota(jnp.int32, _L)

    def issue_dma(j, carry):
        chunk = idx_v[pl.ds(idx_base + j * _L, _L)]
        for l in range(_L):
            s = jnp.sum(jnp.where(lanes == l, chunk, 0))
            pltpu.async_copy(tab_hbm.at[s], out_hbm.at[out_base + j * _L + l],
                             dsem)
        return carry

    def issue_stream(j, carry):
        chunk = idx_v[pl.ds(idx_base + _DCH * _L + j * _L, _L)]
        for l in range(_L):
            s = jnp.sum(jnp.where(lanes == l, chunk, 0))
            pltpu.async_copy(tab_hbm.at[s], buf_v.at[j * _L + l], sem)
        return carry

    lax.fori_loop(0, _DCH, issue_dma, 0)
    lax.fori_loop(0, _SCH, issue_stream, 0)


def _gather_body(uids_hbm, bids_hbm, utab_hbm, btab_hbm, ue_hbm, be_hbm,
                 uidx_v, bidx_v, buf0_v, buf1_v, sem0, sem1, dsem,
                 *, b_per_w):
    wid = lax.axis_index("s") * _NC + lax.axis_index("c")
    base = wid * b_per_w
    half = b_per_w // 2
    ndma = _DCH * _L
    nstr = _SCH * _L
    pltpu.sync_copy(uids_hbm.at[pl.ds(base, b_per_w)], uidx_v)
    pltpu.sync_copy(bids_hbm.at[pl.ds(base, b_per_w)], bidx_v)
    bufs = (buf0_v, buf1_v)
    sems = (sem0, sem1)
    passes = [(utab_hbm, uidx_v, ue_hbm, 0), (utab_hbm, uidx_v, ue_hbm, 1),
              (btab_hbm, bidx_v, be_hbm, 0), (btab_hbm, bidx_v, be_hbm, 1)]

    def drain_store(pi):
        tab, _, out_hbm, h = passes[pi]
        buf, sem = bufs[pi % 2], sems[pi % 2]
        pltpu.make_async_copy(tab.at[pl.ds(0, nstr)], buf, sem).wait()
        pltpu.sync_copy(
            buf, out_hbm.at[pl.ds(base + h * half + ndma, nstr)])

    for pi, (tab, idxv, outh, h) in enumerate(passes):
        _issue_pass(tab, idxv, outh, base + h * half, h * half,
                    bufs[pi % 2], sems[pi % 2], dsem)
        if pi >= 1:
            drain_store(pi - 1)
    drain_store(len(passes) - 1)
    pltpu.make_async_copy(utab_hbm.at[pl.ds(0, 4 * ndma)],
                          ue_hbm.at[pl.ds(0, 4 * ndma)], dsem).wait()


def _sc_gather(user_ids, book_ids, user_table, book_table):
    batch = user_ids.shape[0]
    embed = user_table.shape[1]
    b_per_w = batch // _NW
    mesh = plsc.VectorSubcoreMesh(core_axis_name="c", subcore_axis_name="s")
    k = pl.kernel(
        functools.partial(_gather_body, b_per_w=b_per_w),
        out_type=[
            jax.ShapeDtypeStruct((batch, embed), jnp.float32),
            jax.ShapeDtypeStruct((batch, embed), jnp.float32),
        ],
        mesh=mesh,
        scratch_types=[
            pltpu.VMEM((b_per_w,), jnp.int32),
            pltpu.VMEM((b_per_w,), jnp.int32),
            pltpu.VMEM((_SCH * _L, embed), jnp.float32),
            pltpu.VMEM((_SCH * _L, embed), jnp.float32),
            pltpu.SemaphoreType.DMA,
            pltpu.SemaphoreType.DMA,
            pltpu.SemaphoreType.DMA,
        ],
        compiler_params=pltpu.CompilerParams(needs_layout_passes=False),
    )
    return k(user_ids, book_ids, user_table, book_table)


def _silu(x):
    return x * (1.0 / (1.0 + jnp.exp(-x)))


def _mlp_body(ue_ref, be_ref, W1_ref, b1_ref, W2_ref, b2_ref, W3_ref, b3_ref,
              o_ref):
    u = ue_ref[...]
    v = be_ref[...]
    W1 = W1_ref[...]
    e = u.shape[1]
    h = jnp.dot(u, W1[:e], preferred_element_type=jnp.float32)
    h += jnp.dot(v, W1[e:], preferred_element_type=jnp.float32)
    h += b1_ref[...]
    h = _silu(h)
    h = jnp.dot(h, W2_ref[...], preferred_element_type=jnp.float32)
    h += b2_ref[...]
    h = _silu(h)
    o = jnp.dot(h, W3_ref[...], preferred_element_type=jnp.float32)
    o += b3_ref[...]
    o_ref[...] = jnp.maximum(o, 0.0)


def _tc_mlp(ue, be, W1, b1, W2, b2, W3, b3):
    batch, embed = ue.shape
    blk = 2048
    grid = (batch // blk,)
    full = lambda shape: pl.BlockSpec(shape, lambda i: (0, 0))
    return pl.pallas_call(
        _mlp_body,
        grid=grid,
        in_specs=[
            pl.BlockSpec((blk, embed), lambda i: (i, 0)),
            pl.BlockSpec((blk, embed), lambda i: (i, 0)),
            full(W1.shape),
            full((1, b1.shape[0])),
            full(W2.shape),
            full((1, b2.shape[0])),
            full(W3.shape),
            full((1, 1)),
        ],
        out_specs=pl.BlockSpec((blk, 1), lambda i: (i, 0)),
        out_shape=jax.ShapeDtypeStruct((batch, 1), jnp.float32),
    )(ue, be, W1, b1.reshape(1, -1), W2, b2.reshape(1, -1), W3,
      b3.reshape(1, 1))


def kernel(user_ids, book_ids, user_table, book_table, W1, b1, W2, b2, W3, b3):
    ue, be = _sc_gather(user_ids.astype(jnp.int32), book_ids.astype(jnp.int32),
                        user_table, book_table)
    return _tc_mlp(ue, be, W1, b1, W2, b2, W3, b3)


# speedup vs baseline: 1.2418x; 1.2418x over previous
import functools

import jax
import jax.numpy as jnp
from jax import lax
from jax.experimental import pallas as pl
from jax.experimental.pallas import tpu as pltpu
from jax.experimental.pallas import tpu_sc as plsc

_NC = 2
_NS = 16
_NW = _NC * _NS
_L = 16
_S = 32


_NSEM = 4


def _issue_pass(tab_hbm, idx_v, idx_base, buf_v, sems, n_rows):
    lanes = lax.iota(jnp.int32, _L)

    def issue(j, carry):
        chunk = idx_v[pl.ds(idx_base + j * _L, _L)]
        for l in range(_L):
            s = jnp.sum(jnp.where(lanes == l, chunk, 0))
            pltpu.async_copy(tab_hbm.at[s], buf_v.at[j * _L + l],
                             sems[l % _NSEM])
        return carry

    lax.fori_loop(0, n_rows // _L, issue, 0)


def _gather_body(uids_hbm, bids_hbm, utab_hbm, btab_hbm, ue_hbm, be_hbm,
                 uidx_v, bidx_v, buf0_v, buf1_v, sems0, sems1, *, b_per_w):
    wid = lax.axis_index("s") * _NC + lax.axis_index("c")
    base = wid * b_per_w
    half = b_per_w // 2
    pltpu.sync_copy(uids_hbm.at[pl.ds(base, b_per_w)], uidx_v)
    pltpu.sync_copy(bids_hbm.at[pl.ds(base, b_per_w)], bidx_v)
    bufs = (buf0_v, buf1_v)
    semsets = (sems0, sems1)
    passes = [(utab_hbm, uidx_v, ue_hbm, 0), (utab_hbm, uidx_v, ue_hbm, 1),
              (btab_hbm, bidx_v, be_hbm, 0), (btab_hbm, bidx_v, be_hbm, 1)]

    def drain_store(pi):
        tab, _, out_hbm, h = passes[pi]
        buf, sems = bufs[pi % 2], semsets[pi % 2]
        per_sem = half // _NSEM
        for k in range(_NSEM):
            pltpu.make_async_copy(tab.at[pl.ds(0, per_sem)],
                                  buf.at[pl.ds(0, per_sem)], sems[k]).wait()
        pltpu.sync_copy(buf, out_hbm.at[pl.ds(base + h * half, half)])

    for pi, (tab, idxv, _, h) in enumerate(passes):
        _issue_pass(tab, idxv, h * half, bufs[pi % 2], semsets[pi % 2], half)
        if pi >= 1:
            drain_store(pi - 1)
    drain_store(len(passes) - 1)


def _sc_gather(user_ids, book_ids, user_table, book_table):
    batch = user_ids.shape[0]
    embed = user_table.shape[1]
    b_per_w = batch // _NW
    mesh = plsc.VectorSubcoreMesh(core_axis_name="c", subcore_axis_name="s")
    k = pl.kernel(
        functools.partial(_gather_body, b_per_w=b_per_w),
        out_type=[
            jax.ShapeDtypeStruct((batch, embed), jnp.float32),
            jax.ShapeDtypeStruct((batch, embed), jnp.float32),
        ],
        mesh=mesh,
        scratch_types=[
            pltpu.VMEM((b_per_w,), jnp.int32),
            pltpu.VMEM((b_per_w,), jnp.int32),
            pltpu.VMEM((b_per_w // 2, embed), jnp.float32),
            pltpu.VMEM((b_per_w // 2, embed), jnp.float32),
            [pltpu.SemaphoreType.DMA] * _NSEM,
            [pltpu.SemaphoreType.DMA] * _NSEM,
        ],
        compiler_params=pltpu.CompilerParams(needs_layout_passes=False),
    )
    return k(user_ids, book_ids, user_table, book_table)


def _silu(x):
    return x * (1.0 / (1.0 + jnp.exp(-x)))


def _mlp_body(ue_ref, be_ref, W1_ref, b1_ref, W2_ref, b2_ref, W3_ref, b3_ref,
              o_ref):
    u = ue_ref[...]
    v = be_ref[...]
    W1 = W1_ref[...]
    e = u.shape[1]
    h = jnp.dot(u, W1[:e], preferred_element_type=jnp.float32)
    h += jnp.dot(v, W1[e:], preferred_element_type=jnp.float32)
    h += b1_ref[...]
    h = _silu(h)
    h = jnp.dot(h, W2_ref[...], preferred_element_type=jnp.float32)
    h += b2_ref[...]
    h = _silu(h)
    o = jnp.dot(h, W3_ref[...], preferred_element_type=jnp.float32)
    o += b3_ref[...]
    o_ref[...] = jnp.maximum(o, 0.0)


def _tc_mlp(ue, be, W1, b1, W2, b2, W3, b3):
    batch, embed = ue.shape
    blk = 2048
    grid = (batch // blk,)
    full = lambda shape: pl.BlockSpec(shape, lambda i: (0, 0))
    return pl.pallas_call(
        _mlp_body,
        grid=grid,
        in_specs=[
            pl.BlockSpec((blk, embed), lambda i: (i, 0)),
            pl.BlockSpec((blk, embed), lambda i: (i, 0)),
            full(W1.shape),
            full((1, b1.shape[0])),
            full(W2.shape),
            full((1, b2.shape[0])),
            full(W3.shape),
            full((1, 1)),
        ],
        out_specs=pl.BlockSpec((blk, 1), lambda i: (i, 0)),
        out_shape=jax.ShapeDtypeStruct((batch, 1), jnp.float32),
    )(ue, be, W1, b1.reshape(1, -1), W2, b2.reshape(1, -1), W3,
      b3.reshape(1, 1))


def kernel(user_ids, book_ids, user_table, book_table, W1, b1, W2, b2, W3, b3):
    ue, be = _sc_gather(user_ids.astype(jnp.int32), book_ids.astype(jnp.int32),
                        user_table, book_table)
    return _tc_mlp(ue, be, W1, b1, W2, b2, W3, b3)


# speedup vs baseline: 1.2431x; 1.0011x over previous
import functools

import jax
import jax.numpy as jnp
from jax import lax
from jax.experimental import pallas as pl
from jax.experimental.pallas import tpu as pltpu
from jax.experimental.pallas import tpu_sc as plsc

_NC = 2
_NS = 16
_NW = _NC * _NS
_L = 16
_S = 32


_NSEM = 4


def _issue_pass(tab_hbm, idx_v, idx_base, buf_v, sems, n_rows):
    def issue(j, carry):
        chunk = idx_v[pl.ds(idx_base + j * _L, _L)]
        for l in range(_L):
            s = chunk[l]
            pltpu.async_copy(tab_hbm.at[s], buf_v.at[j * _L + l],
                             sems[l % _NSEM])
        return carry

    lax.fori_loop(0, n_rows // _L, issue, 0)


def _gather_body(uids_hbm, bids_hbm, utab_hbm, btab_hbm, ue_hbm, be_hbm,
                 uidx_v, bidx_v, buf0_v, buf1_v, sems0, sems1, *, b_per_w):
    wid = lax.axis_index("s") * _NC + lax.axis_index("c")
    base = wid * b_per_w
    half = b_per_w // 2
    pltpu.sync_copy(uids_hbm.at[pl.ds(base, b_per_w)], uidx_v)
    pltpu.sync_copy(bids_hbm.at[pl.ds(base, b_per_w)], bidx_v)
    bufs = (buf0_v, buf1_v)
    semsets = (sems0, sems1)
    passes = [(utab_hbm, uidx_v, ue_hbm, 0), (utab_hbm, uidx_v, ue_hbm, 1),
              (btab_hbm, bidx_v, be_hbm, 0), (btab_hbm, bidx_v, be_hbm, 1)]

    def drain_store(pi):
        tab, _, out_hbm, h = passes[pi]
        buf, sems = bufs[pi % 2], semsets[pi % 2]
        per_sem = half // _NSEM
        for k in range(_NSEM):
            pltpu.make_async_copy(tab.at[pl.ds(0, per_sem)],
                                  buf.at[pl.ds(0, per_sem)], sems[k]).wait()
        pltpu.sync_copy(buf, out_hbm.at[pl.ds(base + h * half, half)])

    for pi, (tab, idxv, _, h) in enumerate(passes):
        _issue_pass(tab, idxv, h * half, bufs[pi % 2], semsets[pi % 2], half)
        if pi >= 1:
            drain_store(pi - 1)
    drain_store(len(passes) - 1)


def _sc_gather(user_ids, book_ids, user_table, book_table):
    batch = user_ids.shape[0]
    embed = user_table.shape[1]
    b_per_w = batch // _NW
    mesh = plsc.VectorSubcoreMesh(core_axis_name="c", subcore_axis_name="s")
    k = pl.kernel(
        functools.partial(_gather_body, b_per_w=b_per_w),
        out_type=[
            jax.ShapeDtypeStruct((batch, embed), jnp.float32),
            jax.ShapeDtypeStruct((batch, embed), jnp.float32),
        ],
        mesh=mesh,
        scratch_types=[
            pltpu.VMEM((b_per_w,), jnp.int32),
            pltpu.VMEM((b_per_w,), jnp.int32),
            pltpu.VMEM((b_per_w // 2, embed), jnp.float32),
            pltpu.VMEM((b_per_w // 2, embed), jnp.float32),
            [pltpu.SemaphoreType.DMA] * _NSEM,
            [pltpu.SemaphoreType.DMA] * _NSEM,
        ],
        compiler_params=pltpu.CompilerParams(needs_layout_passes=False),
    )
    return k(user_ids, book_ids, user_table, book_table)


def _silu(x):
    return x * (1.0 / (1.0 + jnp.exp(-x)))


def _mlp_body(ue_ref, be_ref, W1_ref, b1_ref, W2_ref, b2_ref, W3_ref, b3_ref,
              o_ref):
    u = ue_ref[...]
    v = be_ref[...]
    W1 = W1_ref[...]
    e = u.shape[1]
    h = jnp.dot(u, W1[:e], preferred_element_type=jnp.float32)
    h += jnp.dot(v, W1[e:], preferred_element_type=jnp.float32)
    h += b1_ref[...]
    h = _silu(h)
    h = jnp.dot(h, W2_ref[...], preferred_element_type=jnp.float32)
    h += b2_ref[...]
    h = _silu(h)
    o = jnp.dot(h, W3_ref[...], preferred_element_type=jnp.float32)
    o += b3_ref[...]
    o_ref[...] = jnp.maximum(o, 0.0)


def _tc_mlp(ue, be, W1, b1, W2, b2, W3, b3):
    batch, embed = ue.shape
    blk = 2048
    grid = (batch // blk,)
    full = lambda shape: pl.BlockSpec(shape, lambda i: (0, 0))
    return pl.pallas_call(
        _mlp_body,
        grid=grid,
        in_specs=[
            pl.BlockSpec((blk, embed), lambda i: (i, 0)),
            pl.BlockSpec((blk, embed), lambda i: (i, 0)),
            full(W1.shape),
            full((1, b1.shape[0])),
            full(W2.shape),
            full((1, b2.shape[0])),
            full(W3.shape),
            full((1, 1)),
        ],
        out_specs=pl.BlockSpec((blk, 1), lambda i: (i, 0)),
        out_shape=jax.ShapeDtypeStruct((batch, 1), jnp.float32),
    )(ue, be, W1, b1.reshape(1, -1), W2, b2.reshape(1, -1), W3,
      b3.reshape(1, 1))


def kernel(user_ids, book_ids, user_table, book_table, W1, b1, W2, b2, W3, b3):
    ue, be = _sc_gather(user_ids.astype(jnp.int32), book_ids.astype(jnp.int32),
                        user_table, book_table)
    return _tc_mlp(ue, be, W1, b1, W2, b2, W3, b3)


# speedup vs baseline: 1.2465x; 1.0027x over previous
import functools

import jax
import jax.numpy as jnp
from jax import lax
from jax.experimental import pallas as pl
from jax.experimental.pallas import tpu as pltpu
from jax.experimental.pallas import tpu_sc as plsc

_NC = 2
_NS = 16
_NW = _NC * _NS
_L = 16
_S = 32


_NSEM = 4


def _issue_pass(tab_hbm, idx_v, idx_base, buf_v, sems, n_rows):
    def issue(j, carry):
        chunk = idx_v[pl.ds(idx_base + j * _L, _L)]
        for l in range(_L):
            s = chunk[l]
            pltpu.async_copy(tab_hbm.at[s], buf_v.at[j * _L + l],
                             sems[l % _NSEM])
        return carry

    lax.fori_loop(0, n_rows // _L, issue, 0)


def _gather_body(uids_hbm, bids_hbm, utab_hbm, btab_hbm, ue_hbm, be_hbm,
                 uidx_v, bidx_v, buf0_v, buf1_v, sems0, sems1, *, b_per_w):
    wid = lax.axis_index("s") * _NC + lax.axis_index("c")
    base = wid * b_per_w
    half = b_per_w // 2
    pltpu.sync_copy(uids_hbm.at[pl.ds(base, b_per_w)], uidx_v)
    pltpu.sync_copy(bids_hbm.at[pl.ds(base, b_per_w)], bidx_v)
    bufs = (buf0_v, buf1_v)
    semsets = (sems0, sems1)
    passes = [(utab_hbm, uidx_v, ue_hbm, 0), (utab_hbm, uidx_v, ue_hbm, 1),
              (btab_hbm, bidx_v, be_hbm, 0), (btab_hbm, bidx_v, be_hbm, 1)]

    def drain_store(pi):
        tab, _, out_hbm, h = passes[pi]
        buf, sems = bufs[pi % 2], semsets[pi % 2]
        per_sem = half // _NSEM
        for k in range(_NSEM):
            pltpu.make_async_copy(tab.at[pl.ds(0, per_sem)],
                                  buf.at[pl.ds(0, per_sem)], sems[k]).wait()
        pltpu.sync_copy(buf, out_hbm.at[pl.ds(base + h * half, half)])

    for pi, (tab, idxv, _, h) in enumerate(passes):
        _issue_pass(tab, idxv, h * half, bufs[pi % 2], semsets[pi % 2], half)
        if pi >= 1:
            drain_store(pi - 1)
    drain_store(len(passes) - 1)


def _sc_gather(user_ids, book_ids, user_table, book_table):
    batch = user_ids.shape[0]
    embed = user_table.shape[1]
    b_per_w = batch // _NW
    mesh = plsc.VectorSubcoreMesh(core_axis_name="c", subcore_axis_name="s")
    k = pl.kernel(
        functools.partial(_gather_body, b_per_w=b_per_w),
        out_type=[
            jax.ShapeDtypeStruct((batch, embed), jnp.float32),
            jax.ShapeDtypeStruct((batch, embed), jnp.float32),
        ],
        mesh=mesh,
        scratch_types=[
            pltpu.VMEM((b_per_w,), jnp.int32),
            pltpu.VMEM((b_per_w,), jnp.int32),
            pltpu.VMEM((b_per_w // 2, embed), jnp.float32),
            pltpu.VMEM((b_per_w // 2, embed), jnp.float32),
            [pltpu.SemaphoreType.DMA] * _NSEM,
            [pltpu.SemaphoreType.DMA] * _NSEM,
        ],
    )
    return k(user_ids, book_ids, user_table, book_table)


def _silu(x):
    return x * (1.0 / (1.0 + jnp.exp(-x)))


def _mlp_body(ue_ref, be_ref, W1_ref, b1_ref, W2_ref, b2_ref, W3_ref, b3_ref,
              o_ref):
    u = ue_ref[...]
    v = be_ref[...]
    W1 = W1_ref[...]
    e = u.shape[1]
    h = jnp.dot(u, W1[:e], preferred_element_type=jnp.float32)
    h += jnp.dot(v, W1[e:], preferred_element_type=jnp.float32)
    h += b1_ref[...]
    h = _silu(h)
    h = jnp.dot(h, W2_ref[...], preferred_element_type=jnp.float32)
    h += b2_ref[...]
    h = _silu(h)
    o = jnp.dot(h, W3_ref[...], preferred_element_type=jnp.float32)
    o += b3_ref[...]
    o_ref[...] = jnp.maximum(o, 0.0)


def _tc_mlp(ue, be, W1, b1, W2, b2, W3, b3):
    batch, embed = ue.shape
    blk = 2048
    grid = (batch // blk,)
    full = lambda shape: pl.BlockSpec(shape, lambda i: (0, 0))
    return pl.pallas_call(
        _mlp_body,
        grid=grid,
        in_specs=[
            pl.BlockSpec((blk, embed), lambda i: (i, 0)),
            pl.BlockSpec((blk, embed), lambda i: (i, 0)),
            full(W1.shape),
            full((1, b1.shape[0])),
            full(W2.shape),
            full((1, b2.shape[0])),
            full(W3.shape),
            full((1, 1)),
        ],
        out_specs=pl.BlockSpec((blk, 1), lambda i: (i, 0)),
        out_shape=jax.ShapeDtypeStruct((batch, 1), jnp.float32),
    )(ue, be, W1, b1.reshape(1, -1), W2, b2.reshape(1, -1), W3,
      b3.reshape(1, 1))


def kernel(user_ids, book_ids, user_table, book_table, W1, b1, W2, b2, W3, b3):
    ue, be = _sc_gather(user_ids.astype(jnp.int32), book_ids.astype(jnp.int32),
                        user_table, book_table)
    return _tc_mlp(ue, be, W1, b1, W2, b2, W3, b3)
